# baseline (speedup 1.0000x reference)
import jax
import jax.numpy as jnp
from jax.experimental import pallas as pl
from jax.experimental.pallas import tpu as pltpu

_D_IN = 1024
_D_H = 512


def _fused_fwd(h_ref, wfc_ref, bfc_ref, wcls_ref, bcls_ref, out_ref):
    x = jnp.dot(
        h_ref[...].astype(jnp.bfloat16),
        wfc_ref[...].astype(jnp.bfloat16),
        preferred_element_type=jnp.float32,
    )
    x = jnp.maximum(x + bfc_ref[...], 0.0)
    out_ref[...] = (
        jnp.dot(x, wcls_ref[...], preferred_element_type=jnp.float32)
        + bcls_ref[...]
    )


def kernel(h, W_fc, b_fc, W_a, b_a, W_b, b_b, W_c, b_c, W_cls, b_cls):
    h = jnp.squeeze(h)
    n, d_in = h.shape
    n_cls = W_cls.shape[1]
    block_m = 1000
    assert n % block_m == 0
    grid = (n // block_m,)
    return pl.pallas_call(
        _fused_fwd,
        grid=grid,
        in_specs=[
            pl.BlockSpec((block_m, d_in), lambda i: (i, 0)),
            pl.BlockSpec((d_in, _D_H), lambda i: (0, 0)),
            pl.BlockSpec((1, _D_H), lambda i: (0, 0)),
            pl.BlockSpec((_D_H, n_cls), lambda i: (0, 0)),
            pl.BlockSpec((1, n_cls), lambda i: (0, 0)),
        ],
        out_specs=pl.BlockSpec((block_m, n_cls), lambda i: (i, 0)),
        out_shape=jax.ShapeDtypeStruct((n, n_cls), jnp.float32),
        compiler_params=pltpu.CompilerParams(
            dimension_semantics=("parallel",),
        ),
    )(h, W_fc, b_fc.reshape(1, _D_H), W_cls, b_cls.reshape(1, n_cls))


# speedup vs baseline: 1.1215x; 1.1215x over previous
import jax
import jax.numpy as jnp
from jax.experimental import pallas as pl
from jax.experimental.pallas import tpu as pltpu

_D_IN = 1024
_D_H = 512


def _fused_fwd(h0_ref, h1_ref, w0_ref, w1_ref, bfc_ref, wcls_ref, bcls_ref,
               out_ref):
    x = jnp.dot(
        h0_ref[...].astype(jnp.bfloat16),
        w0_ref[...].astype(jnp.bfloat16),
        preferred_element_type=jnp.float32,
    )
    x += jnp.dot(
        h1_ref[...].astype(jnp.bfloat16),
        w1_ref[...].astype(jnp.bfloat16),
        preferred_element_type=jnp.float32,
    )
    x = jnp.maximum(x + bfc_ref[...], 0.0)
    out_ref[...] = (
        jnp.dot(x, wcls_ref[...], preferred_element_type=jnp.float32)
        + bcls_ref[...]
    )


def kernel(h, W_fc, b_fc, W_a, b_a, W_b, b_b, W_c, b_c, W_cls, b_cls):
    h = jnp.squeeze(h)
    n, d_in = h.shape
    n_cls = W_cls.shape[1]
    half = d_in // 2
    block_m = 2000
    assert n % block_m == 0
    grid = (n // block_m,)
    return pl.pallas_call(
        _fused_fwd,
        grid=grid,
        in_specs=[
            pl.BlockSpec((block_m, half), lambda i: (i, 0)),
            pl.BlockSpec((block_m, half), lambda i: (i, 1)),
            pl.BlockSpec((half, _D_H), lambda i: (0, 0)),
            pl.BlockSpec((half, _D_H), lambda i: (1, 0)),
            pl.BlockSpec((1, _D_H), lambda i: (0, 0)),
            pl.BlockSpec((_D_H, n_cls), lambda i: (0, 0)),
            pl.BlockSpec((1, n_cls), lambda i: (0, 0)),
        ],
        out_specs=pl.BlockSpec((block_m, n_cls), lambda i: (i, 0)),
        out_shape=jax.ShapeDtypeStruct((n, n_cls), jnp.float32),
        compiler_params=pltpu.CompilerParams(
            dimension_semantics=("parallel",),
        ),
    )(h, h, W_fc, W_fc, b_fc.reshape(1, _D_H), W_cls, b_cls.reshape(1, n_cls))


# speedup vs baseline: 1.1476x; 1.0232x over previous
import jax
import jax.numpy as jnp
from jax.experimental import pallas as pl
from jax.experimental.pallas import tpu as pltpu

_D_IN = 1024
_D_H = 512


def _fused_fwd(h_ref, wfc_ref, bfc_ref, wcls_ref, bcls_ref, o0_ref, o1_ref):
    x = jnp.dot(
        h_ref[...].astype(jnp.bfloat16),
        wfc_ref[...].astype(jnp.bfloat16),
        preferred_element_type=jnp.float32,
    )
    x = jnp.maximum(x + bfc_ref[...], 0.0)
    logits = (
        jnp.dot(x, wcls_ref[...], preferred_element_type=jnp.float32)
        + bcls_ref[...]
    )
    o0_ref[...] = logits[:, 0]
    o1_ref[...] = logits[:, 1]


def kernel(h, W_fc, b_fc, W_a, b_a, W_b, b_b, W_c, b_c, W_cls, b_cls):
    h = jnp.squeeze(h)
    n, d_in = h.shape
    n_cls = W_cls.shape[1]
    block_m = 2048
    grid = (pl.cdiv(n, block_m),)
    o0, o1 = pl.pallas_call(
        _fused_fwd,
        grid=grid,
        in_specs=[
            pl.BlockSpec((block_m, d_in), lambda i: (i, 0)),
            pl.BlockSpec((d_in, _D_H), lambda i: (0, 0)),
            pl.BlockSpec((1, _D_H), lambda i: (0, 0)),
            pl.BlockSpec((_D_H, n_cls), lambda i: (0, 0)),
            pl.BlockSpec((1, n_cls), lambda i: (0, 0)),
        ],
        out_specs=[
            pl.BlockSpec((block_m,), lambda i: (i,)),
            pl.BlockSpec((block_m,), lambda i: (i,)),
        ],
        out_shape=[
            jax.ShapeDtypeStruct((n,), jnp.float32),
            jax.ShapeDtypeStruct((n,), jnp.float32),
        ],
        compiler_params=pltpu.CompilerParams(
            dimension_semantics=("parallel",),
        ),
    )(h, W_fc, b_fc.reshape(1, _D_H), W_cls, b_cls.reshape(1, n_cls))
    return jnp.stack([o0, o1], axis=1)


# speedup vs baseline: 1.3939x; 1.2147x over previous
import jax
import jax.numpy as jnp
from jax.experimental import pallas as pl
from jax.experimental.pallas import tpu as pltpu

_D_IN = 1024
_D_H = 512


def _fused_fwd(h_ref, wfc_ref, bfc_ref, wcls_ref, bclsT_ref, out_ref):
    x = jnp.dot(
        h_ref[...].astype(jnp.bfloat16),
        wfc_ref[...].astype(jnp.bfloat16),
        preferred_element_type=jnp.float32,
    )
    x = jnp.maximum(x + bfc_ref[...], 0.0)
    logits_t = jax.lax.dot_general(
        wcls_ref[...], x,
        dimension_numbers=(((0,), (1,)), ((), ())),
        preferred_element_type=jnp.float32,
    )
    out_ref[...] = logits_t + bclsT_ref[...]


def kernel(h, W_fc, b_fc, W_a, b_a, W_b, b_b, W_c, b_c, W_cls, b_cls):
    h = jnp.squeeze(h)
    n, d_in = h.shape
    n_cls = W_cls.shape[1]
    block_m = 2048
    grid = (pl.cdiv(n, block_m),)
    out_t = pl.pallas_call(
        _fused_fwd,
        grid=grid,
        in_specs=[
            pl.BlockSpec((block_m, d_in), lambda i: (i, 0)),
            pl.BlockSpec((d_in, _D_H), lambda i: (0, 0)),
            pl.BlockSpec((1, _D_H), lambda i: (0, 0)),
            pl.BlockSpec((_D_H, n_cls), lambda i: (0, 0)),
            pl.BlockSpec((n_cls, 1), lambda i: (0, 0)),
        ],
        out_specs=pl.BlockSpec((n_cls, block_m), lambda i: (0, i)),
        out_shape=jax.ShapeDtypeStruct((n_cls, n), jnp.float32),
        compiler_params=pltpu.CompilerParams(
            dimension_semantics=("parallel",),
        ),
    )(h, W_fc, b_fc.reshape(1, _D_H), W_cls, b_cls.reshape(n_cls, 1))
    return out_t.T


# speedup vs baseline: 1.4157x; 1.0157x over previous
import jax
import jax.numpy as jnp
from jax.experimental import pallas as pl
from jax.experimental.pallas import tpu as pltpu

_D_IN = 1024
_D_H = 512


def _fused_fwd(h_ref, wfc_ref, bfc_ref, wcls_ref, bcls_ref, out_ref):
    x = jnp.dot(
        h_ref[...].astype(jnp.bfloat16),
        wfc_ref[...].astype(jnp.bfloat16),
        preferred_element_type=jnp.float32,
    )
    x = jnp.maximum(x + bfc_ref[...], 0.0).astype(jnp.bfloat16)
    logits_t = jax.lax.dot_general(
        wcls_ref[...].astype(jnp.bfloat16), x,
        dimension_numbers=(((0,), (1,)), ((), ())),
        preferred_element_type=jnp.float32,
    )
    out_ref[...] = logits_t + bcls_ref[...].T


def kernel(h, W_fc, b_fc, W_a, b_a, W_b, b_b, W_c, b_c, W_cls, b_cls):
    h = jnp.squeeze(h)
    n, d_in = h.shape
    n_cls = W_cls.shape[1]
    block_m = 2048
    grid = (pl.cdiv(n, block_m),)
    out_t = pl.pallas_call(
        _fused_fwd,
        grid=grid,
        in_specs=[
            pl.BlockSpec((block_m, d_in), lambda i: (i, 0)),
            pl.BlockSpec((d_in, _D_H), lambda i: (0, 0)),
            pl.BlockSpec((1, _D_H), lambda i: (0, 0)),
            pl.BlockSpec((_D_H, n_cls), lambda i: (0, 0)),
            pl.BlockSpec((1, n_cls), lambda i: (0, 0)),
        ],
        out_specs=pl.BlockSpec((n_cls, block_m), lambda i: (0, i)),
        out_shape=jax.ShapeDtypeStruct((n_cls, n), jnp.float32),
        compiler_params=pltpu.CompilerParams(
            dimension_semantics=("parallel",),
        ),
    )(h, W_fc, b_fc.reshape(1, _D_H), W_cls, b_cls.reshape(1, n_cls))
    return out_t.T


# speedup vs baseline: 1.4366x; 1.0147x over previous
import jax
import jax.numpy as jnp
from jax.experimental import pallas as pl
from jax.experimental.pallas import tpu as pltpu

_D_IN = 1024
_D_H = 512


def _fused_fwd(h_ref, wfc_ref, bfc_ref, wcls_ref, bcls_ref, out_ref):
    x = jnp.dot(
        h_ref[...].astype(jnp.bfloat16),
        wfc_ref[...].astype(jnp.bfloat16),
        preferred_element_type=jnp.float32,
    )
    x = jnp.maximum(x + bfc_ref[...], 0.0).astype(jnp.bfloat16)
    logits_t = jax.lax.dot_general(
        wcls_ref[...].astype(jnp.bfloat16), x,
        dimension_numbers=(((0,), (1,)), ((), ())),
        preferred_element_type=jnp.float32,
    )
    out_ref[...] = logits_t + bcls_ref[...].T


def kernel(h, W_fc, b_fc, W_a, b_a, W_b, b_b, W_c, b_c, W_cls, b_cls):
    h = jnp.squeeze(h)
    n, d_in = h.shape
    n_cls = W_cls.shape[1]
    block_m = 4096
    grid = (pl.cdiv(n, block_m),)
    out_t = pl.pallas_call(
        _fused_fwd,
        grid=grid,
        in_specs=[
            pl.BlockSpec((block_m, d_in), lambda i: (i, 0)),
            pl.BlockSpec((d_in, _D_H), lambda i: (0, 0)),
            pl.BlockSpec((1, _D_H), lambda i: (0, 0)),
            pl.BlockSpec((_D_H, n_cls), lambda i: (0, 0)),
            pl.BlockSpec((1, n_cls), lambda i: (0, 0)),
        ],
        out_specs=pl.BlockSpec((n_cls, block_m), lambda i: (0, i)),
        out_shape=jax.ShapeDtypeStruct((n_cls, n), jnp.float32),
        compiler_params=pltpu.CompilerParams(
            dimension_semantics=("parallel",),
        ),
    )(h, W_fc, b_fc.reshape(1, _D_H), W_cls, b_cls.reshape(1, n_cls))
    return out_t.T


# speedup vs baseline: 1.4759x; 1.0274x over previous
import jax
import jax.numpy as jnp
from jax.experimental import pallas as pl
from jax.experimental.pallas import tpu as pltpu

_D_IN = 1024
_D_H = 512


def _fused_fwd(h_ref, wfc_ref, bfc_ref, wcls_ref, bcls_ref, out_ref):
    x = jnp.dot(
        h_ref[...].astype(jnp.bfloat16),
        wfc_ref[...].astype(jnp.bfloat16),
        preferred_element_type=jnp.float32,
    )
    x = jnp.maximum(x + bfc_ref[...], 0.0).astype(jnp.bfloat16)
    logits_t = jax.lax.dot_general(
        wcls_ref[...].astype(jnp.bfloat16), x,
        dimension_numbers=(((0,), (1,)), ((), ())),
        preferred_element_type=jnp.float32,
    )
    out_ref[...] = logits_t + bcls_ref[...].T


def kernel(h, W_fc, b_fc, W_a, b_a, W_b, b_b, W_c, b_c, W_cls, b_cls):
    h = jnp.squeeze(h)
    n, d_in = h.shape
    n_cls = W_cls.shape[1]
    block_m = 5120
    grid = (pl.cdiv(n, block_m),)
    out_t = pl.pallas_call(
        _fused_fwd,
        grid=grid,
        in_specs=[
            pl.BlockSpec((block_m, d_in), lambda i: (i, 0)),
            pl.BlockSpec((d_in, _D_H), lambda i: (0, 0)),
            pl.BlockSpec((1, _D_H), lambda i: (0, 0)),
            pl.BlockSpec((_D_H, n_cls), lambda i: (0, 0)),
            pl.BlockSpec((1, n_cls), lambda i: (0, 0)),
        ],
        out_specs=pl.BlockSpec((n_cls, block_m), lambda i: (0, i)),
        out_shape=jax.ShapeDtypeStruct((n_cls, n), jnp.float32),
        compiler_params=pltpu.CompilerParams(
            dimension_semantics=("parallel",),
        ),
    )(h, W_fc, b_fc.reshape(1, _D_H), W_cls, b_cls.reshape(1, n_cls))
    return out_t.T
